# baseline (speedup 1.0000x reference)
import functools

import jax
import jax.numpy as jnp
from jax import lax
from jax.experimental import pallas as pl
from jax.experimental.pallas import tpu as pltpu
from jax.experimental.pallas import tpu_sc as plsc

_NU = 25000
_NI = 25000
_D = 64
_NE = 800000
_N2 = _NU + _NI
_HALF = 32
_B = 128
_ACC_ROWS = 51200
_DUMP = 50000
_NC, _NS = 2, 16

@functools.lru_cache(maxsize=None)
def _mesh():
  return plsc.VectorSubcoreMesh(
      core_axis_name="c", subcore_axis_name="s",
      num_cores=_NC, num_subcores=_NS)


_CH = 256
_NSUB = _CH // _B
_EPT = 50432
_NCH = _EPT // _CH


@functools.lru_cache(maxsize=None)
def _make_spmm(remap):
  rows_per_tile = _ACC_ROWS // _NS
  orows = 3128
  olast = _N2 - (_NS - 1) * orows
  bcast_dn = lax.GatherDimensionNumbers(
      offset_dims=(), collapsed_slice_dims=(0,), start_index_map=(0,))

  @functools.partial(
      pl.kernel,
      out_type=[jax.ShapeDtypeStruct((_N2, _HALF), jnp.float32)] * 2,
      mesh=_mesh(),
      compiler_params=pltpu.CompilerParams(use_tc_tiling_on_sc=False),
      scratch_types=[
          pltpu.VMEM((_CH,), jnp.int32),
          pltpu.VMEM((_CH,), jnp.int32),
          pltpu.VMEM((2, _CH), jnp.float32),
          pltpu.VMEM((2, _NSUB, _B), jnp.int32),
          pltpu.VMEM((2, _CH, _HALF), jnp.float32),
          pltpu.VMEM_SHARED((_ACC_ROWS, _HALF), jnp.float32),
          pltpu.SemaphoreType.DMA,
          pltpu.SemaphoreType.DMA,
          pltpu.SemaphoreType.DMA,
      ],
  )
  def spmm(dst_h, src_h, val_h, x0_h, x1_h, zero_h, y0_h, y1_h,
           dstb, srcb, valb, ldst, rows, acc, sem_l, sem_g, sem_s):
    c = lax.axis_index("c")
    s = lax.axis_index("s")

    pltpu.sync_copy(zero_h, rows.at[0, pl.ds(0, _B), :])
    def zcopy(k, carry):
      pltpu.sync_copy(rows.at[0, pl.ds(0, _B), :],
                      acc.at[pl.ds(s * rows_per_tile + k * _B, _B), :])
      return carry
    lax.fori_loop(0, rows_per_tile // _B, zcopy, 0)
    plsc.subcore_barrier()

    e0 = s * _EPT

    def load_lists(k1, p):
      off = e0 + k1 * _CH
      return [
          pltpu.async_copy(dst_h.at[pl.ds(off, _CH)], dstb, sem_l),
          pltpu.async_copy(src_h.at[pl.ds(off, _CH)], srcb, sem_l),
          pltpu.async_copy(val_h.at[pl.ds(off, _CH)], valb.at[p], sem_l),
      ]

    def remap_to(p):
      for g in range(_NSUB * 8):
        d = dstb[pl.ds(g * 16, 16)]
        if remap:
          d = jnp.where(d < _NU, d,
                        jnp.where(d >= 3 * _NU, d - 2 * _NU,
                                  jnp.full((16,), _DUMP, jnp.int32)))
        ldst[p, g // 8, pl.ds((g % 8) * 16, 16)] = d

    def fire_gathers(p):
      def fire(x_h):
        for j in range(_NSUB):
          pltpu.async_copy(x_h.at[srcb.at[pl.ds(j * _B, _B)]],
                           rows.at[p, pl.ds(j * _B, _B), :], sem_g)
      @pl.when(c == 0)
      def _():
        fire(x0_h)
      @pl.when(c == 1)
      def _():
        fire(x1_h)

    def wait_gathers(p):
      def wt(x_h):
        for j in range(_NSUB):
          pltpu.make_async_copy(x_h.at[srcb.at[pl.ds(j * _B, _B)]],
                                rows.at[p, pl.ds(j * _B, _B), :], sem_g).wait()
      @pl.when(c == 0)
      def _():
        wt(x0_h)
      @pl.when(c == 1)
      def _():
        wt(x1_h)

    def fire_scatters(p):
      for j in range(_NSUB):
        pltpu.async_copy(rows.at[p, pl.ds(j * _B, _B), :],
                         acc.at[ldst.at[p, j]], sem_s, add=True)

    def drain_scatters(p):
      for j in range(_NSUB):
        pltpu.make_async_copy(rows.at[p, pl.ds(j * _B, _B), :],
                              acc.at[ldst.at[p, j]], sem_s).wait()

    def scale(p):
      for j in range(_NSUB):
        rblk = rows.at[p, pl.ds(j * _B, _B), :]
        vblk = valb.at[p, pl.ds(j * _B, _B)]
        for g in range(_B // 16):
          vv = vblk[pl.ds(g * 16, 16)]
          for t in range(16):
            e = g * 16 + t
            b = lax.gather(vv, jnp.full((16, 1), t, jnp.int32), bcast_dn,
                           (1,), mode=lax.GatherScatterMode.PROMISE_IN_BOUNDS)
            rblk[e, pl.ds(0, 16)] = rblk[e, pl.ds(0, 16)] * b
            rblk[e, pl.ds(16, 16)] = rblk[e, pl.ds(16, 16)] * b

    def steady(k, p):
      ls = load_lists(k + 1, 1 - p)
      drain_scatters(1 - p)
      for l in ls:
        l.wait()
      remap_to(1 - p)
      fire_gathers(1 - p)
      wait_gathers(p)
      scale(p)
      fire_scatters(p)

    for j in range(_NSUB):
      pltpu.sync_copy(zero_h, rows.at[1, pl.ds(j * _B, _B), :])
      for g in range(8):
        ldst[1, j, pl.ds(g * 16, 16)] = jnp.full((16,), _DUMP, jnp.int32)
    fire_scatters(1)
    for l in load_lists(0, 0):
      l.wait()
    remap_to(0)
    fire_gathers(0)

    def pair(i, carry):
      steady(2 * i, 0)
      steady(2 * i + 1, 1)
      return carry
    lax.fori_loop(0, (_NCH - 1) // 2, pair, 0)

    drain_scatters(1)
    wait_gathers(0)
    scale(0)
    fire_scatters(0)
    drain_scatters(0)

    plsc.subcore_barrier()

    def owrite(y_h):
      r0 = pl.multiple_of(s * orows, 8)

      @pl.when(s < _NS - 1)
      def _():
        pltpu.sync_copy(acc.at[pl.ds(r0, orows), :], y_h.at[pl.ds(r0, orows), :])

      @pl.when(s == _NS - 1)
      def _():
        pltpu.sync_copy(acc.at[pl.ds((_NS - 1) * orows, olast), :],
                        y_h.at[pl.ds((_NS - 1) * orows, olast), :])

    @pl.when(c == 0)
    def _():
      owrite(y0_h)

    @pl.when(c == 1)
    def _():
      owrite(y1_h)

  return spmm


def _pad_edges(dst, src, vals):
  per = _NE // _NS
  pad = _EPT - per
  d = jnp.pad(dst.reshape(_NS, per), ((0, 0), (0, pad)),
              constant_values=_DUMP).reshape(-1)
  s_ = jnp.pad(src.reshape(_NS, per), ((0, 0), (0, pad))).reshape(-1)
  v = jnp.pad(vals.reshape(_NS, per), ((0, 0), (0, pad))).reshape(-1)
  return d, s_, v


@functools.lru_cache(maxsize=None)
def _make_gather12():
  @functools.partial(
      pl.kernel,
      out_type=[jax.ShapeDtypeStruct((2048, _D), jnp.float32)] * 12,
      mesh=_mesh(),
      compiler_params=pltpu.CompilerParams(use_tc_tiling_on_sc=False),
      scratch_types=[
          pltpu.VMEM((64,), jnp.int32),
          pltpu.VMEM((64,), jnp.int32),
          pltpu.VMEM((64, _D), jnp.float32),
          pltpu.SemaphoreType.DMA,
      ],
  )
  def gather12(users_h, pos_h, neg_h, ut_h, it_h, y1_h, y2_h, y3_h,
               gu0, gu1, gu2, gu3, gp0, gp1, gp2, gp3, gn0, gn1, gn2, gn3,
               idxb, idxb2, buf, sem):
    c = lax.axis_index("c")
    s = lax.axis_index("s")
    wid = s * _NC + c
    base = wid * 64

    def fetch(src, idx_ref, out):
      pltpu.async_copy(src.at[idx_ref], buf, sem).wait()
      pltpu.sync_copy(buf, out.at[pl.ds(base, 64), :])

    pltpu.sync_copy(users_h.at[pl.ds(base, 64)], idxb)
    fetch(ut_h, idxb, gu0)
    fetch(y1_h, idxb, gu1)
    fetch(y2_h, idxb, gu2)
    fetch(y3_h, idxb, gu3)

    for idx_h, o0, o1, o2, o3 in ((pos_h, gp0, gp1, gp2, gp3),
                                  (neg_h, gn0, gn1, gn2, gn3)):
      pltpu.sync_copy(idx_h.at[pl.ds(base, 64)], idxb)
      for g in range(4):
        idxb2[pl.ds(g * 16, 16)] = idxb[pl.ds(g * 16, 16)] + _NU
      fetch(it_h, idxb, o0)
      fetch(y1_h, idxb2, o1)
      fetch(y2_h, idxb2, o2)
      fetch(y3_h, idxb2, o3)

  return gather12


def _att_body(n_real, x_ref, att_ref, adj_ref, out_ref):
  x = x_ref[...]
  a = att_ref[...]
  xn = x / jnp.maximum(jnp.sqrt(jnp.sum(x * x, axis=1, keepdims=True)), 1e-12)
  an = a / jnp.maximum(jnp.sqrt(jnp.sum(a * a, axis=1, keepdims=True)), 1e-12)
  sim = lax.dot_general(xn, an, (((1,), (1,)), ((), ())),
                        preferred_element_type=jnp.float32)
  sim = sim * adj_ref[...]
  col = lax.broadcasted_iota(jnp.int32, sim.shape, 1)
  masked = jnp.where(col >= n_real, -3e16, jnp.where(sim != 0.0, sim, -9e15))
  m = jnp.max(masked, axis=1, keepdims=True)
  e = jnp.exp(masked - m)
  att = e / jnp.sum(e, axis=1, keepdims=True)
  out_ref[...] = jnp.dot(att, a, preferred_element_type=jnp.float32)


def _make_att(n_rows, n_real, n_att_pad, blk=1000):
  return pl.pallas_call(
      functools.partial(_att_body, n_real),
      grid=(n_rows // blk,),
      in_specs=[
          pl.BlockSpec((blk, _D), lambda i: (i, 0)),
          pl.BlockSpec((n_att_pad, _D), lambda i: (0, 0)),
          pl.BlockSpec((blk, n_att_pad), lambda i: (i, 0)),
      ],
      out_specs=pl.BlockSpec((blk, _D), lambda i: (i, 0)),
      out_shape=jax.ShapeDtypeStruct((n_rows, _D), jnp.float32),
  )


_att_u = _make_att(_NU, 10, 16)
_att_i = _make_att(_NI, 35, 48)


def _loss_body(gu0, gu1, gu2, gu3, gp0, gp1, gp2, gp3, gn0, gn1, gn2, gn3,
               loss_ref, reg_ref):
  u0 = gu0[...]
  p0 = gp0[...]
  n0 = gn0[...]
  lu = (u0 + gu1[...] + gu2[...] + gu3[...]) * 0.25
  lp = (p0 + gp1[...] + gp2[...] + gp3[...]) * 0.25
  ln = (n0 + gn1[...] + gn2[...] + gn3[...]) * 0.25
  pos_s = jnp.sum(lu * lp, axis=1)
  neg_s = jnp.sum(lu * ln, axis=1)
  x = neg_s - pos_s
  sp = jnp.maximum(x, 0.0) + jnp.log1p(jnp.exp(-jnp.abs(x)))
  loss_ref[0, 0] = jnp.mean(sp)
  reg_ref[0, 0] = 0.5 * (jnp.sum(u0 * u0) + jnp.sum(p0 * p0)
                         + jnp.sum(n0 * n0)) / 2048.0


_loss_call = pl.pallas_call(
    _loss_body,
    out_shape=[jax.ShapeDtypeStruct((1, 1), jnp.float32)] * 2,
    out_specs=[pl.BlockSpec(memory_space=pltpu.SMEM)] * 2,
)


def kernel(users, pos, neg, user_table, item_table, uatt1_table, uatt2_table,
           iatt1_table, iatt2_table, iatt3_table, graph_idx, graph_vals,
           graph_att_idx, graph_att_vals, user_att_adj, item_att_adj):
  users = users.astype(jnp.int32)
  pos = pos.astype(jnp.int32)
  neg = neg.astype(jnp.int32)
  gidx = graph_idx.astype(jnp.int32)
  gaidx = graph_att_idx.astype(jnp.int32)

  uatt = jnp.concatenate([uatt1_table, uatt2_table], axis=0)
  iatt = jnp.concatenate([iatt1_table, iatt2_table, iatt3_table], axis=0)
  uatt_p = jnp.pad(uatt, ((0, 6), (0, 0)))
  iatt_p = jnp.pad(iatt, ((0, 13), (0, 0)))
  uadj_p = jnp.pad(user_att_adj, ((0, 0), (0, 6)))
  iadj_p = jnp.pad(item_att_adj, ((0, 0), (0, 13)))

  uatt_e = _att_u(user_table, uatt_p, uadj_p)
  iatt_e = _att_i(item_table, iatt_p, iadj_p)

  zeros_blk = jnp.zeros((_B, _HALF), jnp.float32)
  x0 = jnp.concatenate([user_table, iatt_e, uatt_e, item_table], axis=0)
  spmm_remap = _make_spmm(True)
  spmm_plain = _make_spmm(False)
  gad, gas, gav = _pad_edges(gaidx[0], gaidx[1], graph_att_vals)
  gd, gs, gv = _pad_edges(gidx[0], gidx[1], graph_vals)
  y1a, y1b = spmm_remap(gad, gas, gav,
                        x0[:, :_HALF], x0[:, _HALF:], zeros_blk)
  y2a, y2b = spmm_plain(gd, gs, gv, y1a, y1b, zeros_blk)
  y3a, y3b = spmm_plain(gd, gs, gv, y2a, y2b, zeros_blk)
  y1 = jnp.concatenate([y1a, y1b], axis=1)
  y2 = jnp.concatenate([y2a, y2b], axis=1)
  y3 = jnp.concatenate([y3a, y3b], axis=1)

  g = _make_gather12()(users, pos, neg, user_table, item_table, y1, y2, y3)
  loss, reg = _loss_call(*g)
  return loss[0, 0], reg[0, 0]


# speedup vs baseline: 1.0176x; 1.0176x over previous
import functools

import jax
import jax.numpy as jnp
from jax import lax
from jax.experimental import pallas as pl
from jax.experimental.pallas import tpu as pltpu
from jax.experimental.pallas import tpu_sc as plsc

_NU = 25000
_NI = 25000
_D = 64
_NE = 800000
_N2 = _NU + _NI
_HALF = 32
_B = 128
_ACC_ROWS = 51200
_DUMP = 50000
_NC, _NS = 2, 16

@functools.lru_cache(maxsize=None)
def _mesh():
  return plsc.VectorSubcoreMesh(
      core_axis_name="c", subcore_axis_name="s",
      num_cores=_NC, num_subcores=_NS)


_CH = 640
_NSUB = _CH // _B


@functools.lru_cache(maxsize=None)
def _make_spmm(remap):
  e_per_tile = _NE // _NS
  nch = e_per_tile // _CH
  tail = e_per_tile - nch * _CH
  rows_per_tile = _ACC_ROWS // _NS
  orows = 3128
  olast = _N2 - (_NS - 1) * orows
  bcast_dn = lax.GatherDimensionNumbers(
      offset_dims=(), collapsed_slice_dims=(0,), start_index_map=(0,))

  @functools.partial(
      pl.kernel,
      out_type=[jax.ShapeDtypeStruct((_N2, _HALF), jnp.float32)] * 2,
      mesh=_mesh(),
      compiler_params=pltpu.CompilerParams(use_tc_tiling_on_sc=False),
      scratch_types=[
          pltpu.VMEM((_CH,), jnp.int32),
          pltpu.VMEM((_CH,), jnp.int32),
          pltpu.VMEM((_CH,), jnp.float32),
          pltpu.VMEM((_NSUB, _B), jnp.int32),
          pltpu.VMEM((_CH, _HALF), jnp.float32),
          pltpu.VMEM_SHARED((_ACC_ROWS, _HALF), jnp.float32),
          pltpu.SemaphoreType.DMA,
          pltpu.SemaphoreType.DMA,
          pltpu.SemaphoreType.DMA,
      ],
  )
  def spmm(dst_h, src_h, val_h, x0_h, x1_h, zero_h, y0_h, y1_h,
           dstb, srcb, valb, ldst, rows, acc, sem_l, sem_g, sem_s):
    c = lax.axis_index("c")
    s = lax.axis_index("s")

    pltpu.sync_copy(zero_h, rows.at[pl.ds(0, _B), :])
    def zcopy(k, carry):
      pltpu.sync_copy(rows.at[pl.ds(0, _B), :],
                      acc.at[pl.ds(s * rows_per_tile + k * _B, _B), :])
      return carry
    lax.fori_loop(0, rows_per_tile // _B, zcopy, 0)
    plsc.subcore_barrier()

    e0 = s * e_per_tile

    def remap_group(g):
      d = dstb[pl.ds(g * 16, 16)]
      if remap:
        d = jnp.where(d < _NU, d,
                      jnp.where(d >= 3 * _NU, d - 2 * _NU,
                                jnp.full((16,), _DUMP, jnp.int32)))
      ldst[g // 8, pl.ds((g % 8) * 16, 16)] = d

    def scale_block(j, carry):
      rblk = rows.at[pl.ds(j * _B, _B), :]
      vblk = valb.at[pl.ds(j * _B, _B)]
      for g in range(_B // 16):
        vv = vblk[pl.ds(g * 16, 16)]
        for t in range(16):
          e = g * 16 + t
          b = lax.gather(vv, jnp.full((16, 1), t, jnp.int32), bcast_dn, (1,),
                         mode=lax.GatherScatterMode.PROMISE_IN_BOUNDS)
          rblk[e, pl.ds(0, 16)] = rblk[e, pl.ds(0, 16)] * b
          rblk[e, pl.ds(16, 16)] = rblk[e, pl.ds(16, 16)] * b
      return carry

    def process(x_h, off, nb):
      l1 = pltpu.async_copy(dst_h.at[pl.ds(off, nb)], dstb.at[pl.ds(0, nb)],
                            sem_l)
      l2 = pltpu.async_copy(src_h.at[pl.ds(off, nb)], srcb.at[pl.ds(0, nb)],
                            sem_l)
      l3 = pltpu.async_copy(val_h.at[pl.ds(off, nb)], valb.at[pl.ds(0, nb)],
                            sem_l)
      l1.wait()
      l2.wait()
      l3.wait()
      nsub = nb // _B
      if nb % _B:
        for g in range(nb // 16, (nsub + 1) * 8):
          srcb[pl.ds(g * 16, 16)] = jnp.zeros((16,), jnp.int32)
          valb[pl.ds(g * 16, 16)] = jnp.zeros((16,), jnp.float32)
          dstb[pl.ds(g * 16, 16)] = jnp.full((16,), _DUMP, jnp.int32)
        nsub += 1
      for g in range(nsub * 8):
        remap_group(g)
      gathers = [
          pltpu.async_copy(x_h.at[srcb.at[pl.ds(j * _B, _B)]],
                           rows.at[pl.ds(j * _B, _B), :], sem_g)
          for j in range(nsub)
      ]
      scatters = []
      for j in range(nsub):
        gathers[j].wait()
        scale_block(j, 0)
        scatters.append(
            pltpu.async_copy(rows.at[pl.ds(j * _B, _B), :],
                             acc.at[ldst.at[j]], sem_s, add=True))
      for sd in scatters:
        sd.wait()

    def run_edges(x_h):
      def blk(k, carry):
        process(x_h, e0 + k * _CH, _CH)
        return carry
      lax.fori_loop(0, nch, blk, 0)
      process(x_h, e0 + nch * _CH, tail)

    @pl.when(c == 0)
    def _():
      run_edges(x0_h)

    @pl.when(c == 1)
    def _():
      run_edges(x1_h)

    plsc.subcore_barrier()

    def owrite(y_h):
      r0 = pl.multiple_of(s * orows, 8)

      @pl.when(s < _NS - 1)
      def _():
        pltpu.sync_copy(acc.at[pl.ds(r0, orows), :], y_h.at[pl.ds(r0, orows), :])

      @pl.when(s == _NS - 1)
      def _():
        pltpu.sync_copy(acc.at[pl.ds((_NS - 1) * orows, olast), :],
                        y_h.at[pl.ds((_NS - 1) * orows, olast), :])

    @pl.when(c == 0)
    def _():
      owrite(y0_h)

    @pl.when(c == 1)
    def _():
      owrite(y1_h)

  return spmm


@functools.lru_cache(maxsize=None)
def _make_gather12():
  @functools.partial(
      pl.kernel,
      out_type=[jax.ShapeDtypeStruct((2048, _D), jnp.float32)] * 12,
      mesh=_mesh(),
      compiler_params=pltpu.CompilerParams(use_tc_tiling_on_sc=False),
      scratch_types=[
          pltpu.VMEM((64,), jnp.int32),
          pltpu.VMEM((64,), jnp.int32),
          pltpu.VMEM((64, _D), jnp.float32),
          pltpu.SemaphoreType.DMA,
      ],
  )
  def gather12(users_h, pos_h, neg_h, ut_h, it_h, y1_h, y2_h, y3_h,
               gu0, gu1, gu2, gu3, gp0, gp1, gp2, gp3, gn0, gn1, gn2, gn3,
               idxb, idxb2, buf, sem):
    c = lax.axis_index("c")
    s = lax.axis_index("s")
    wid = s * _NC + c
    base = wid * 64

    def fetch(src, idx_ref, out):
      pltpu.async_copy(src.at[idx_ref], buf, sem).wait()
      pltpu.sync_copy(buf, out.at[pl.ds(base, 64), :])

    pltpu.sync_copy(users_h.at[pl.ds(base, 64)], idxb)
    fetch(ut_h, idxb, gu0)
    fetch(y1_h, idxb, gu1)
    fetch(y2_h, idxb, gu2)
    fetch(y3_h, idxb, gu3)

    for idx_h, o0, o1, o2, o3 in ((pos_h, gp0, gp1, gp2, gp3),
                                  (neg_h, gn0, gn1, gn2, gn3)):
      pltpu.sync_copy(idx_h.at[pl.ds(base, 64)], idxb)
      for g in range(4):
        idxb2[pl.ds(g * 16, 16)] = idxb[pl.ds(g * 16, 16)] + _NU
      fetch(it_h, idxb, o0)
      fetch(y1_h, idxb2, o1)
      fetch(y2_h, idxb2, o2)
      fetch(y3_h, idxb2, o3)

  return gather12


def _att_body(n_real, x_ref, att_ref, adj_ref, out_ref):
  x = x_ref[...]
  a = att_ref[...]
  xn = x / jnp.maximum(jnp.sqrt(jnp.sum(x * x, axis=1, keepdims=True)), 1e-12)
  an = a / jnp.maximum(jnp.sqrt(jnp.sum(a * a, axis=1, keepdims=True)), 1e-12)
  sim = lax.dot_general(xn, an, (((1,), (1,)), ((), ())),
                        preferred_element_type=jnp.float32)
  sim = sim * adj_ref[...]
  col = lax.broadcasted_iota(jnp.int32, sim.shape, 1)
  masked = jnp.where(col >= n_real, -3e16, jnp.where(sim != 0.0, sim, -9e15))
  m = jnp.max(masked, axis=1, keepdims=True)
  e = jnp.exp(masked - m)
  att = e / jnp.sum(e, axis=1, keepdims=True)
  out_ref[...] = jnp.dot(att, a, preferred_element_type=jnp.float32)


def _make_att(n_rows, n_real, n_att_pad, blk=1000):
  return pl.pallas_call(
      functools.partial(_att_body, n_real),
      grid=(n_rows // blk,),
      in_specs=[
          pl.BlockSpec((blk, _D), lambda i: (i, 0)),
          pl.BlockSpec((n_att_pad, _D), lambda i: (0, 0)),
          pl.BlockSpec((blk, n_att_pad), lambda i: (i, 0)),
      ],
      out_specs=pl.BlockSpec((blk, _D), lambda i: (i, 0)),
      out_shape=jax.ShapeDtypeStruct((n_rows, _D), jnp.float32),
  )


_att_u = _make_att(_NU, 10, 16)
_att_i = _make_att(_NI, 35, 48)


def _loss_body(gu0, gu1, gu2, gu3, gp0, gp1, gp2, gp3, gn0, gn1, gn2, gn3,
               loss_ref, reg_ref):
  u0 = gu0[...]
  p0 = gp0[...]
  n0 = gn0[...]
  lu = (u0 + gu1[...] + gu2[...] + gu3[...]) * 0.25
  lp = (p0 + gp1[...] + gp2[...] + gp3[...]) * 0.25
  ln = (n0 + gn1[...] + gn2[...] + gn3[...]) * 0.25
  pos_s = jnp.sum(lu * lp, axis=1)
  neg_s = jnp.sum(lu * ln, axis=1)
  x = neg_s - pos_s
  sp = jnp.maximum(x, 0.0) + jnp.log1p(jnp.exp(-jnp.abs(x)))
  loss_ref[0, 0] = jnp.mean(sp)
  reg_ref[0, 0] = 0.5 * (jnp.sum(u0 * u0) + jnp.sum(p0 * p0)
                         + jnp.sum(n0 * n0)) / 2048.0


_loss_call = pl.pallas_call(
    _loss_body,
    out_shape=[jax.ShapeDtypeStruct((1, 1), jnp.float32)] * 2,
    out_specs=[pl.BlockSpec(memory_space=pltpu.SMEM)] * 2,
)


def kernel(users, pos, neg, user_table, item_table, uatt1_table, uatt2_table,
           iatt1_table, iatt2_table, iatt3_table, graph_idx, graph_vals,
           graph_att_idx, graph_att_vals, user_att_adj, item_att_adj):
  users = users.astype(jnp.int32)
  pos = pos.astype(jnp.int32)
  neg = neg.astype(jnp.int32)
  gidx = graph_idx.astype(jnp.int32)
  gaidx = graph_att_idx.astype(jnp.int32)

  uatt = jnp.concatenate([uatt1_table, uatt2_table], axis=0)
  iatt = jnp.concatenate([iatt1_table, iatt2_table, iatt3_table], axis=0)
  uatt_p = jnp.pad(uatt, ((0, 6), (0, 0)))
  iatt_p = jnp.pad(iatt, ((0, 13), (0, 0)))
  uadj_p = jnp.pad(user_att_adj, ((0, 0), (0, 6)))
  iadj_p = jnp.pad(item_att_adj, ((0, 0), (0, 13)))

  uatt_e = _att_u(user_table, uatt_p, uadj_p)
  iatt_e = _att_i(item_table, iatt_p, iadj_p)

  zeros_blk = jnp.zeros((_B, _HALF), jnp.float32)
  x0 = jnp.concatenate([user_table, iatt_e, uatt_e, item_table], axis=0)
  spmm_remap = _make_spmm(True)
  spmm_plain = _make_spmm(False)
  y1a, y1b = spmm_remap(gaidx[0], gaidx[1], graph_att_vals,
                        x0[:, :_HALF], x0[:, _HALF:], zeros_blk)
  y2a, y2b = spmm_plain(gidx[0], gidx[1], graph_vals, y1a, y1b, zeros_blk)
  y3a, y3b = spmm_plain(gidx[0], gidx[1], graph_vals, y2a, y2b, zeros_blk)
  y1 = jnp.concatenate([y1a, y1b], axis=1)
  y2 = jnp.concatenate([y2a, y2b], axis=1)
  y3 = jnp.concatenate([y3a, y3b], axis=1)

  g = _make_gather12()(users, pos, neg, user_table, item_table, y1, y2, y3)
  loss, reg = _loss_call(*g)
  return loss[0, 0], reg[0, 0]
